# baseline (speedup 1.0000x reference)
import functools

import jax
import jax.numpy as jnp
from jax import lax
from jax.experimental import pallas as pl
from jax.experimental.pallas import tpu as pltpu
from jax.experimental.pallas import tpu_sc as plsc

_VOCAB = 1000000
_C_SIZE = 1000
_EMBED = 64
_B = 16384
_L = 50

_NC = 2
_NS = 16
_NW = _NC * _NS

_N = _B * _L
_BPW = _B // _NW
_NBH = _BPW // 128
_TP = 513

_mesh = plsc.VectorSubcoreMesh(core_axis_name="c", subcore_axis_name="s")


@functools.partial(
    pl.kernel,
    mesh=_mesh,
    compiler_params=pltpu.CompilerParams(
        use_tc_tiling_on_sc=False, needs_layout_passes=False),
    out_type=[
        jax.ShapeDtypeStruct((_L, 8, _B // 128, 8, 128), jnp.float32),
        jax.ShapeDtypeStruct((8, _B // 128, 8, 128), jnp.float32),
    ],
    scratch_types=[
        pltpu.VMEM((_BPW,), jnp.int32),
        pltpu.VMEM((_BPW,), jnp.int32),
        pltpu.VMEM((_BPW,), jnp.int32),
        pltpu.VMEM((_BPW, _EMBED), jnp.float32),
        pltpu.VMEM((_BPW, _EMBED), jnp.float32),
        pltpu.VMEM((_EMBED, _TP), jnp.float32),
        pltpu.SemaphoreType.DMA,
        pltpu.SemaphoreType.DMA,
        pltpu.SemaphoreType.DMA,
    ],
)
def _encode(xt_hbm, c_hbm, s_tab, c_tab, out_s, out_c,
            c_v, idx0, idx1, rows0, rows1, t_v, g0, g1, s0):
    wid = lax.axis_index("s") * _NC + lax.axis_index("c")
    idxs = (idx0, idx1)
    rows = (rows0, rows1)
    gsems = (g0, g1)

    iota = jax.lax.iota(jnp.int32, 16)
    e_rows = [iota + 16 * e0 for e0 in range(4)]

    pltpu.sync_copy(c_hbm.at[pl.ds(wid * _BPW, _BPW)], c_v)

    def load_idx(l, p):
        pltpu.sync_copy(xt_hbm.at[pl.ds(l * _B + wid * _BPW, _BPW)], idxs[p])

    def fire_gather(p):
        pltpu.async_copy(s_tab.at[idxs[p]], rows[p], gsems[p])

    def wait_gather(p):
        pltpu.make_async_copy(s_tab.at[idxs[p]], rows[p], gsems[p]).wait()

    def transpose(rows_ref, nb):
        @pl.loop(0, nb, unroll=8)
        def _t(b):
            col = jnp.full((16,), b, dtype=jnp.int32)
            for e0 in range(4):
                v = rows_ref[b, pl.ds(16 * e0, 16)]
                plsc.store_scatter(t_v, [e_rows[e0], col], v)

    def s_tiles(l):
        return [(t_v.at[pl.ds(8 * eh, 8), pl.ds(128 * bl, 128)],
                 out_s.at[l, eh, _NBH * wid + bl])
                for bl in range(_NBH) for eh in range(8)]

    def fire_s_stores(l):
        for src, dst in s_tiles(l):
            pltpu.async_copy(src, dst, s0)

    def wait_s_stores(l):
        for src, dst in s_tiles(l):
            pltpu.make_async_copy(src, dst, s0).wait()

    load_idx(0, 0)
    fire_gather(0)
    load_idx(1, 1)
    fire_gather(1)

    @pl.loop(0, _L, step=2)
    def _units(g):
        for p in range(2):
            l = g + p
            wait_gather(p)

            @pl.when(l >= 1)
            def _():
                wait_s_stores(l - 1)

            transpose(rows[p], _BPW)
            fire_s_stores(l)

            @pl.when(l + 2 < _L)
            def _():
                load_idx(l + 2, p)
                fire_gather(p)

    wait_s_stores(_L - 1)

    @pl.loop(0, _NBH)
    def _cunits(bl):
        pltpu.async_copy(c_tab.at[c_v.at[pl.ds(128 * bl, 128)]],
                         rows0.at[pl.ds(0, 128)], g0).wait()
        transpose(rows0, 128)
        bh = _NBH * wid + bl
        for eh in range(8):
            pltpu.async_copy(t_v.at[pl.ds(8 * eh, 8), pl.ds(0, 128)],
                             out_c.at[eh, bh], s0)
        for eh in range(8):
            pltpu.make_async_copy(t_v.at[pl.ds(8 * eh, 8), pl.ds(0, 128)],
                                  out_c.at[eh, bh], s0).wait()


def kernel(inputs_x, inputs_c, s_table, c_table):
    xt_flat = inputs_x.T.reshape(_N)
    out_s5, out_c4 = _encode(xt_flat, inputs_c, s_table, c_table)
    out_s = out_s5.transpose(2, 4, 0, 1, 3).reshape(_B, _L, _EMBED)
    out_c = out_c4.transpose(1, 3, 0, 2).reshape(_B, _EMBED)
    return out_s, out_c


# speedup vs baseline: 1.0302x; 1.0302x over previous
import functools

import jax
import jax.numpy as jnp
from jax import lax
from jax.experimental import pallas as pl
from jax.experimental.pallas import tpu as pltpu
from jax.experimental.pallas import tpu_sc as plsc

_VOCAB = 1000000
_C_SIZE = 1000
_EMBED = 64
_B = 16384
_L = 50

_NC = 2
_NS = 16
_NW = _NC * _NS

_N = _B * _L
_BPW = _B // _NW
_NBH = _BPW // 128
_TP = 513

_mesh = plsc.VectorSubcoreMesh(core_axis_name="c", subcore_axis_name="s")


@functools.partial(
    pl.kernel,
    mesh=_mesh,
    compiler_params=pltpu.CompilerParams(
        use_tc_tiling_on_sc=False, needs_layout_passes=False),
    out_type=[
        jax.ShapeDtypeStruct((_L, 8, _B // 128, 8, 128), jnp.float32),
        jax.ShapeDtypeStruct((8, _B // 128, 8, 128), jnp.float32),
    ],
    scratch_types=[
        pltpu.VMEM((_BPW,), jnp.int32),
        pltpu.VMEM((_L, _BPW), jnp.int32),
        pltpu.VMEM((_BPW, _EMBED), jnp.float32),
        pltpu.VMEM((_BPW, _EMBED), jnp.float32),
        pltpu.VMEM((_EMBED, _TP), jnp.float32),
        pltpu.SemaphoreType.DMA,
        pltpu.SemaphoreType.DMA,
        pltpu.SemaphoreType.DMA,
    ],
)
def _encode(xt_hbm, c_hbm, s_tab, c_tab, out_s, out_c,
            c_v, x_v, rows0, rows1, t_v, g0, g1, s0):
    wid = lax.axis_index("s") * _NC + lax.axis_index("c")
    rows = (rows0, rows1)
    gsems = (g0, g1)

    iota = jax.lax.iota(jnp.int32, 16)
    e_rows = [iota + 16 * e0 for e0 in range(4)]

    pltpu.sync_copy(c_hbm.at[pl.ds(wid * _BPW, _BPW)], c_v)
    pltpu.sync_copy(xt_hbm.at[:, pl.ds(wid * _BPW, _BPW)], x_v)

    def fire_gather(l, p):
        pltpu.async_copy(s_tab.at[x_v.at[l]], rows[p], gsems[p])

    def wait_gather(l, p):
        pltpu.make_async_copy(s_tab.at[x_v.at[l]], rows[p], gsems[p]).wait()

    def transpose(rows_ref, nb):
        @pl.loop(0, nb, unroll=16)
        def _t(b):
            col = jnp.full((16,), b, dtype=jnp.int32)
            for e0 in range(4):
                v = rows_ref[b, pl.ds(16 * e0, 16)]
                plsc.store_scatter(t_v, [e_rows[e0], col], v)

    def s_tiles(l):
        return [(t_v.at[pl.ds(8 * eh, 8), pl.ds(128 * bl, 128)],
                 out_s.at[l, eh, _NBH * wid + bl])
                for bl in range(_NBH) for eh in range(8)]

    def fire_s_stores(l):
        for src, dst in s_tiles(l):
            pltpu.async_copy(src, dst, s0)

    def wait_s_stores(l):
        for src, dst in s_tiles(l):
            pltpu.make_async_copy(src, dst, s0).wait()

    fire_gather(0, 0)
    fire_gather(1, 1)

    @pl.loop(0, _L, step=2)
    def _units(g):
        for p in range(2):
            l = g + p
            wait_gather(l, p)

            @pl.when(l >= 1)
            def _():
                wait_s_stores(l - 1)

            transpose(rows[p], _BPW)
            fire_s_stores(l)

            @pl.when(l + 2 < _L)
            def _():
                fire_gather(l + 2, p)

    wait_s_stores(_L - 1)

    @pl.loop(0, _NBH)
    def _cunits(bl):
        pltpu.async_copy(c_tab.at[c_v.at[pl.ds(128 * bl, 128)]],
                         rows0.at[pl.ds(0, 128)], g0).wait()
        transpose(rows0, 128)
        bh = _NBH * wid + bl
        for eh in range(8):
            pltpu.async_copy(t_v.at[pl.ds(8 * eh, 8), pl.ds(0, 128)],
                             out_c.at[eh, bh], s0)
        for eh in range(8):
            pltpu.make_async_copy(t_v.at[pl.ds(8 * eh, 8), pl.ds(0, 128)],
                                  out_c.at[eh, bh], s0).wait()


def kernel(inputs_x, inputs_c, s_table, c_table):
    xt = inputs_x.T
    out_s5, out_c4 = _encode(xt, inputs_c, s_table, c_table)
    out_s = out_s5.transpose(2, 4, 0, 1, 3).reshape(_B, _L, _EMBED)
    out_c = out_c4.transpose(1, 3, 0, 2).reshape(_B, _EMBED)
    return out_s, out_c


# speedup vs baseline: 1.4049x; 1.3638x over previous
import functools

import jax
import jax.numpy as jnp
from jax import lax
from jax.experimental import pallas as pl
from jax.experimental.pallas import tpu as pltpu
from jax.experimental.pallas import tpu_sc as plsc

_VOCAB = 1000000
_C_SIZE = 1000
_EMBED = 64
_B = 16384
_L = 50

_NC = 2
_NS = 16
_NW = _NC * _NS

_N = _B * _L
_BPW = _B // _NW
_NBH = _BPW // 128
_TP = 513

_mesh = plsc.VectorSubcoreMesh(core_axis_name="c", subcore_axis_name="s")


@functools.partial(
    pl.kernel,
    mesh=_mesh,
    compiler_params=pltpu.CompilerParams(
        use_tc_tiling_on_sc=False, needs_layout_passes=False),
    out_type=[
        jax.ShapeDtypeStruct((_L, 8, _B // 128, 8, 128), jnp.float32),
        jax.ShapeDtypeStruct((8, _B // 128, 8, 128), jnp.float32),
    ],
    scratch_types=[
        pltpu.VMEM((_BPW,), jnp.int32),
        pltpu.VMEM((_L, _BPW), jnp.int32),
        pltpu.VMEM((_BPW, _EMBED), jnp.float32),
        pltpu.VMEM((_BPW, _EMBED), jnp.float32),
        pltpu.VMEM((_EMBED, _TP), jnp.float32),
        pltpu.SemaphoreType.DMA,
        pltpu.SemaphoreType.DMA,
        pltpu.SemaphoreType.DMA,
    ],
)
def _encode(xt_hbm, c_hbm, s_tab, c_tab, out_s, out_c,
            c_v, x_v, rows0, rows1, t_v, g0, g1, s0):
    wid = lax.axis_index("s") * _NC + lax.axis_index("c")
    rows = (rows0, rows1)
    gsems = (g0, g1)

    iota = jax.lax.iota(jnp.int32, 16)
    e_rows = [iota + 16 * e0 for e0 in range(4)]

    pltpu.sync_copy(c_hbm.at[pl.ds(wid * _BPW, _BPW)], c_v)
    pltpu.sync_copy(xt_hbm.at[:, pl.ds(wid * _BPW, _BPW)], x_v)

    def fire_gather(l, p):
        pltpu.async_copy(s_tab.at[x_v.at[l]], rows[p], gsems[p])

    def wait_gather(l, p):
        pltpu.make_async_copy(s_tab.at[x_v.at[l]], rows[p], gsems[p]).wait()

    def transpose(rows_ref, nb):
        @plsc.parallel_loop(0, nb, unroll=16)
        def _t(b):
            col = jnp.full((16,), b, dtype=jnp.int32)
            for e0 in range(4):
                v = rows_ref[b, pl.ds(16 * e0, 16)]
                plsc.store_scatter(t_v, [e_rows[e0], col], v)

    def s_tiles(l):
        return [(t_v.at[pl.ds(8 * eh, 8), pl.ds(128 * bl, 128)],
                 out_s.at[l, eh, _NBH * wid + bl])
                for bl in range(_NBH) for eh in range(8)]

    def fire_s_stores(l):
        for src, dst in s_tiles(l):
            pltpu.async_copy(src, dst, s0)

    def wait_s_stores(l):
        for src, dst in s_tiles(l):
            pltpu.make_async_copy(src, dst, s0).wait()

    fire_gather(0, 0)
    fire_gather(1, 1)

    @pl.loop(0, _L, step=2)
    def _units(g):
        for p in range(2):
            l = g + p
            wait_gather(l, p)

            @pl.when(l >= 1)
            def _():
                wait_s_stores(l - 1)

            transpose(rows[p], _BPW)
            fire_s_stores(l)

            @pl.when(l + 2 < _L)
            def _():
                fire_gather(l + 2, p)

    wait_s_stores(_L - 1)

    @pl.loop(0, _NBH)
    def _cunits(bl):
        pltpu.async_copy(c_tab.at[c_v.at[pl.ds(128 * bl, 128)]],
                         rows0.at[pl.ds(0, 128)], g0).wait()
        transpose(rows0, 128)
        bh = _NBH * wid + bl
        for eh in range(8):
            pltpu.async_copy(t_v.at[pl.ds(8 * eh, 8), pl.ds(0, 128)],
                             out_c.at[eh, bh], s0)
        for eh in range(8):
            pltpu.make_async_copy(t_v.at[pl.ds(8 * eh, 8), pl.ds(0, 128)],
                                  out_c.at[eh, bh], s0).wait()


def kernel(inputs_x, inputs_c, s_table, c_table):
    xt = inputs_x.T
    out_s5, out_c4 = _encode(xt, inputs_c, s_table, c_table)
    out_s = out_s5.transpose(2, 4, 0, 1, 3).reshape(_B, _L, _EMBED)
    out_c = out_c4.transpose(1, 3, 0, 2).reshape(_B, _EMBED)
    return out_s, out_c


# speedup vs baseline: 1.4052x; 1.0002x over previous
import functools

import jax
import jax.numpy as jnp
from jax import lax
from jax.experimental import pallas as pl
from jax.experimental.pallas import tpu as pltpu
from jax.experimental.pallas import tpu_sc as plsc

_VOCAB = 1000000
_C_SIZE = 1000
_EMBED = 64
_B = 16384
_L = 50

_NC = 2
_NS = 16
_NW = _NC * _NS

_N = _B * _L
_BPW = _B // _NW
_NBH = _BPW // 128
_TP = 513

_mesh = plsc.VectorSubcoreMesh(core_axis_name="c", subcore_axis_name="s")


@functools.partial(
    pl.kernel,
    mesh=_mesh,
    compiler_params=pltpu.CompilerParams(
        use_tc_tiling_on_sc=False, needs_layout_passes=False),
    out_type=[
        jax.ShapeDtypeStruct((_L, 8, _B // 128, 8, 128), jnp.float32),
        jax.ShapeDtypeStruct((8, _B // 128, 8, 128), jnp.float32),
    ],
    scratch_types=[
        pltpu.VMEM((_BPW,), jnp.int32),
        pltpu.VMEM((_L, _BPW), jnp.int32),
        pltpu.VMEM((_BPW, _EMBED), jnp.float32),
        pltpu.VMEM((_BPW, _EMBED), jnp.float32),
        pltpu.VMEM((_EMBED, _TP), jnp.float32),
        pltpu.SemaphoreType.DMA,
        pltpu.SemaphoreType.DMA,
        pltpu.SemaphoreType.DMA,
    ],
)
def _encode(xt_hbm, c_hbm, s_tab, c_tab, out_s, out_c,
            c_v, x_v, rows0, rows1, t_v, g0, g1, s0):
    wid = lax.axis_index("s") * _NC + lax.axis_index("c")
    rows = (rows0, rows1)
    gsems = (g0, g1)

    iota = jax.lax.iota(jnp.int32, 16)
    e_rows = [iota + 16 * e0 for e0 in range(4)]

    pltpu.sync_copy(c_hbm.at[pl.ds(wid * _BPW, _BPW)], c_v)
    pltpu.sync_copy(xt_hbm.at[:, pl.ds(wid * _BPW, _BPW)], x_v)

    def gather_pairs(l, p):
        h = _BPW // 2
        return [(s_tab.at[x_v.at[l].at[pl.ds(k * h, h)]],
                 rows[p].at[pl.ds(k * h, h)]) for k in range(2)]

    def fire_gather(l, p):
        for src, dst in gather_pairs(l, p):
            pltpu.async_copy(src, dst, gsems[p])

    def wait_gather(l, p):
        for src, dst in gather_pairs(l, p):
            pltpu.make_async_copy(src, dst, gsems[p]).wait()

    def transpose(rows_ref, nb):
        @plsc.parallel_loop(0, nb, unroll=16)
        def _t(b):
            col = jnp.full((16,), b, dtype=jnp.int32)
            for e0 in range(4):
                v = rows_ref[b, pl.ds(16 * e0, 16)]
                plsc.store_scatter(t_v, [e_rows[e0], col], v)

    def s_tiles(l):
        return [(t_v.at[pl.ds(8 * eh, 8), pl.ds(128 * bl, 128)],
                 out_s.at[l, eh, _NBH * wid + bl])
                for bl in range(_NBH) for eh in range(8)]

    def fire_s_stores(l):
        for src, dst in s_tiles(l):
            pltpu.async_copy(src, dst, s0)

    def wait_s_stores(l):
        for src, dst in s_tiles(l):
            pltpu.make_async_copy(src, dst, s0).wait()

    fire_gather(0, 0)
    fire_gather(1, 1)

    @pl.loop(0, _L, step=2)
    def _units(g):
        for p in range(2):
            l = g + p
            wait_gather(l, p)

            @pl.when(l >= 1)
            def _():
                wait_s_stores(l - 1)

            transpose(rows[p], _BPW)
            fire_s_stores(l)

            @pl.when(l + 2 < _L)
            def _():
                fire_gather(l + 2, p)

    wait_s_stores(_L - 1)

    @pl.loop(0, _NBH)
    def _cunits(bl):
        pltpu.async_copy(c_tab.at[c_v.at[pl.ds(128 * bl, 128)]],
                         rows0.at[pl.ds(0, 128)], g0).wait()
        transpose(rows0, 128)
        bh = _NBH * wid + bl
        for eh in range(8):
            pltpu.async_copy(t_v.at[pl.ds(8 * eh, 8), pl.ds(0, 128)],
                             out_c.at[eh, bh], s0)
        for eh in range(8):
            pltpu.make_async_copy(t_v.at[pl.ds(8 * eh, 8), pl.ds(0, 128)],
                                  out_c.at[eh, bh], s0).wait()


def kernel(inputs_x, inputs_c, s_table, c_table):
    xt = inputs_x.T
    out_s5, out_c4 = _encode(xt, inputs_c, s_table, c_table)
    out_s = out_s5.transpose(2, 4, 0, 1, 3).reshape(_B, _L, _EMBED)
    out_c = out_c4.transpose(1, 3, 0, 2).reshape(_B, _EMBED)
    return out_s, out_c


# speedup vs baseline: 1.4144x; 1.0066x over previous
import functools

import jax
import jax.numpy as jnp
from jax import lax
from jax.experimental import pallas as pl
from jax.experimental.pallas import tpu as pltpu
from jax.experimental.pallas import tpu_sc as plsc

_VOCAB = 1000000
_C_SIZE = 1000
_EMBED = 64
_B = 16384
_L = 50

_NC = 2
_NS = 16
_NW = _NC * _NS

_N = _B * _L
_BPW = _B // _NW
_NBH = _BPW // 128
_TP = 513

_mesh = plsc.VectorSubcoreMesh(core_axis_name="c", subcore_axis_name="s")


@functools.partial(
    pl.kernel,
    mesh=_mesh,
    compiler_params=pltpu.CompilerParams(
        use_tc_tiling_on_sc=False, needs_layout_passes=False),
    out_type=[
        jax.ShapeDtypeStruct((_L, 8, _B // 128, 8, 128), jnp.float32),
        jax.ShapeDtypeStruct((8, _B // 128, 8, 128), jnp.float32),
    ],
    scratch_types=[
        pltpu.VMEM((_BPW,), jnp.int32),
        pltpu.VMEM((_L, _BPW), jnp.int32),
        pltpu.VMEM((_BPW, _EMBED), jnp.float32),
        pltpu.VMEM((_BPW, _EMBED), jnp.float32),
        pltpu.VMEM((_EMBED, _TP), jnp.float32),
        pltpu.SemaphoreType.DMA,
        pltpu.SemaphoreType.DMA,
        pltpu.SemaphoreType.DMA,
    ],
)
def _encode(xt_hbm, c_hbm, s_tab, c_tab, out_s, out_c,
            c_v, x_v, rows0, rows1, t_v, g0, g1, s0):
    wid = lax.axis_index("s") * _NC + lax.axis_index("c")
    rows = (rows0, rows1)
    gsems = (g0, g1)

    iota = jax.lax.iota(jnp.int32, 16)
    e_rows = [iota + 16 * e0 for e0 in range(4)]

    pltpu.sync_copy(c_hbm.at[pl.ds(wid * _BPW, _BPW)], c_v)
    pltpu.sync_copy(xt_hbm.at[:, pl.ds(wid * _BPW, _BPW)], x_v)

    def fire_gather(l, p):
        pltpu.async_copy(s_tab.at[x_v.at[l]], rows[p], gsems[p])

    def wait_gather(l, p):
        pltpu.make_async_copy(s_tab.at[x_v.at[l]], rows[p], gsems[p]).wait()

    def transpose(rows_ref, nb):
        @plsc.parallel_loop(0, nb, unroll=16)
        def _t(b):
            col = jnp.full((16,), b, dtype=jnp.int32)
            for e0 in range(4):
                v = rows_ref[b, pl.ds(16 * e0, 16)]
                plsc.store_scatter(t_v, [e_rows[e0], col], v)

    def s_tiles(l):
        return [(t_v.at[pl.ds(8 * eh, 8), pl.ds(128 * bl, 128)],
                 out_s.at[l, eh, _NBH * wid + bl])
                for bl in range(_NBH) for eh in range(8)]

    def fire_s_stores(l):
        for src, dst in s_tiles(l):
            pltpu.async_copy(src, dst, s0)

    def wait_s_stores(l):
        for src, dst in s_tiles(l):
            pltpu.make_async_copy(src, dst, s0).wait()

    fire_gather(0, 0)
    fire_gather(1, 1)

    @pl.loop(0, _L, step=2)
    def _units(g):
        for p in range(2):
            l = g + p
            wait_gather(l, p)

            @pl.when(l >= 1)
            def _():
                wait_s_stores(l - 1)

            transpose(rows[p], _BPW)
            fire_s_stores(l)

            @pl.when(l + 2 < _L)
            def _():
                fire_gather(l + 2, p)

    wait_s_stores(_L - 1)

    @pl.loop(0, _NBH)
    def _cunits(bl):
        pltpu.async_copy(c_tab.at[c_v.at[pl.ds(128 * bl, 128)]],
                         rows0.at[pl.ds(0, 128)], g0).wait()
        transpose(rows0, 128)
        bh = _NBH * wid + bl
        for eh in range(8):
            pltpu.async_copy(t_v.at[pl.ds(8 * eh, 8), pl.ds(0, 128)],
                             out_c.at[eh, bh], s0)
        for eh in range(8):
            pltpu.make_async_copy(t_v.at[pl.ds(8 * eh, 8), pl.ds(0, 128)],
                                  out_c.at[eh, bh], s0).wait()


def kernel(inputs_x, inputs_c, s_table, c_table):
    xt = inputs_x.T
    out_s5, out_c4 = _encode(xt, inputs_c, s_table, c_table)
    out_s = out_s5.transpose(2, 4, 0, 1, 3).reshape(_B, _L, _EMBED)
    out_c = out_c4.transpose(1, 3, 0, 2).reshape(_B, _EMBED)
    return out_s, out_c
